# baseline (device time: 155097 ns/iter reference)
import jax
import jax.numpy as jnp
from jax import lax
from jax.experimental import pallas as pl
from jax.experimental.pallas import tpu as pltpu

N_DEV = 4


def kernel(x, w_mat):
    k, n = w_mat.shape
    m = x.shape[0]
    m_per = m // N_DEV

    def body(x_ref, w_ref, out_ref, acc_ref, send_sems, recv_sems):
        my = lax.axis_index("i")
        left = lax.rem(my + (N_DEV - 1), N_DEV)
        right = lax.rem(my + 1, N_DEV)

        barrier_sem = pltpu.get_barrier_semaphore()
        for nbr in (left, right):
            pl.semaphore_signal(
                barrier_sem, inc=1,
                device_id=(nbr,), device_id_type=pl.DeviceIdType.MESH,
            )
        pl.semaphore_wait(barrier_sem, 2)

        w = w_ref[...].astype(jnp.bfloat16)

        def partial(c):
            xs = x_ref[pl.ds(c * m_per, m_per), :].astype(jnp.bfloat16)
            return jnp.dot(xs, w, preferred_element_type=jnp.float32)

        acc_ref[N_DEV - 1, :, :] = partial(lax.rem(my + N_DEV - 1, N_DEV))

        send_slot = N_DEV - 1
        for h in range(N_DEV - 1):
            rdma = pltpu.make_async_remote_copy(
                src_ref=acc_ref.at[send_slot],
                dst_ref=acc_ref.at[h],
                send_sem=send_sems.at[h],
                recv_sem=recv_sems.at[h],
                device_id=(right,),
                device_id_type=pl.DeviceIdType.MESH,
            )
            rdma.start()
            rdma.wait()
            c = lax.rem(my + N_DEV - 2 - h, N_DEV)
            if h < N_DEV - 2:
                acc_ref[h, :, :] = acc_ref[h, :, :] + partial(c)
            else:
                out_ref[...] = jnp.maximum(acc_ref[h, :, :] + partial(c), 0.0)
            send_slot = h

    return pl.pallas_call(
        body,
        out_shape=jax.ShapeDtypeStruct((m_per, n), jnp.float32),
        in_specs=[
            pl.BlockSpec(memory_space=pltpu.VMEM),
            pl.BlockSpec(memory_space=pltpu.VMEM),
        ],
        out_specs=pl.BlockSpec(memory_space=pltpu.VMEM),
        scratch_shapes=[
            pltpu.VMEM((N_DEV, m_per, n), jnp.float32),
            pltpu.SemaphoreType.DMA((N_DEV - 1,)),
            pltpu.SemaphoreType.DMA((N_DEV - 1,)),
        ],
        compiler_params=pltpu.CompilerParams(collective_id=0),
    )(x, w_mat)


# device time: 50958 ns/iter; 3.0436x vs baseline; 3.0436x over previous
import jax
import jax.numpy as jnp
from jax import lax
from jax.experimental import pallas as pl
from jax.experimental.pallas import tpu as pltpu

N_DEV = 4
N_HOP = N_DEV - 1


def kernel(x, w_mat):
    k, n = w_mat.shape
    m = x.shape[0]
    m_per = m // N_DEV
    n_half = n // 2

    def body(
        x_ref, w_ref, out_ref,
        send_a, send_b, recv_a, recv_b,
        pstore_a, pstore_b,
        ss_a, rs_a, ss_b, rs_b,
    ):
        my = lax.axis_index("i")
        left = lax.rem(my + (N_DEV - 1), N_DEV)
        right = lax.rem(my + 1, N_DEV)

        barrier_sem = pltpu.get_barrier_semaphore()
        for nbr in (left, right):
            pl.semaphore_signal(
                barrier_sem, inc=1,
                device_id=(nbr,), device_id_type=pl.DeviceIdType.MESH,
            )
        pl.semaphore_wait(barrier_sem, 2)

        w = w_ref[...].astype(jnp.bfloat16)

        def partial(c, lo=None):
            xs = x_ref[pl.ds(c * m_per, m_per), :].astype(jnp.bfloat16)
            ws = w if lo is None else w[:, lo:lo + n_half]
            return jnp.dot(xs, ws, preferred_element_type=jnp.float32)

        c_dm1 = lax.rem(my + N_DEV - 1, N_DEV)
        c_dp1 = lax.rem(my + 1, N_DEV)
        c_dp2 = lax.rem(my + 2, N_DEV)

        def hop(h, tgt, src_ref, dst_ref, ssem, rsem):
            return pltpu.make_async_remote_copy(
                src_ref=src_ref.at[h], dst_ref=dst_ref.at[h],
                send_sem=ssem.at[h], recv_sem=rsem.at[h],
                device_id=(tgt,), device_id_type=pl.DeviceIdType.MESH,
            )

        send_a[0, :, :] = partial(c_dm1, 0).astype(jnp.bfloat16)
        send_b[0, :, :] = partial(c_dp1, n_half).astype(jnp.bfloat16)
        r0a = hop(0, right, send_a, recv_a, ss_a, rs_a)
        r0b = hop(0, left, send_b, recv_b, ss_b, rs_b)
        r0a.start()
        r0b.start()

        pstore_a[:, :] = partial(c_dp1, 0).astype(jnp.bfloat16)
        pstore_b[:, :] = partial(c_dm1, n_half).astype(jnp.bfloat16)
        pc = partial(c_dp2)
        pd = partial(my)

        r0a.wait()
        r0b.wait()
        send_a[1, :, :] = (
            recv_a[0, :, :].astype(jnp.float32) + pc[:, :n_half]
        ).astype(jnp.bfloat16)
        send_b[1, :, :] = (
            recv_b[0, :, :].astype(jnp.float32) + pc[:, n_half:]
        ).astype(jnp.bfloat16)

        r1a = hop(1, right, send_a, recv_a, ss_a, rs_a)
        r1b = hop(1, left, send_b, recv_b, ss_b, rs_b)
        r1a.start()
        r1b.start()
        r1a.wait()
        r1b.wait()
        send_a[2, :, :] = (
            recv_a[1, :, :].astype(jnp.float32)
            + pstore_a[:, :].astype(jnp.float32)
        ).astype(jnp.bfloat16)
        send_b[2, :, :] = (
            recv_b[1, :, :].astype(jnp.float32)
            + pstore_b[:, :].astype(jnp.float32)
        ).astype(jnp.bfloat16)

        r2a = hop(2, right, send_a, recv_a, ss_a, rs_a)
        r2b = hop(2, left, send_b, recv_b, ss_b, rs_b)
        r2a.start()
        r2b.start()
        r2a.wait()
        r2b.wait()
        out_ref[:, :n_half] = jnp.maximum(
            recv_a[2, :, :].astype(jnp.float32) + pd[:, :n_half], 0.0
        )
        out_ref[:, n_half:] = jnp.maximum(
            recv_b[2, :, :].astype(jnp.float32) + pd[:, n_half:], 0.0
        )

    comm = pltpu.VMEM((N_HOP, m_per, n_half), jnp.bfloat16)
    half = pltpu.VMEM((m_per, n_half), jnp.bfloat16)
    sems = pltpu.SemaphoreType.DMA((N_HOP,))
    return pl.pallas_call(
        body,
        out_shape=jax.ShapeDtypeStruct((m_per, n), jnp.float32),
        in_specs=[
            pl.BlockSpec(memory_space=pltpu.VMEM),
            pl.BlockSpec(memory_space=pltpu.VMEM),
        ],
        out_specs=pl.BlockSpec(memory_space=pltpu.VMEM),
        scratch_shapes=[comm, comm, comm, comm, half, half,
                        sems, sems, sems, sems],
        compiler_params=pltpu.CompilerParams(collective_id=0),
    )(x, w_mat)


# device time: 46307 ns/iter; 3.3493x vs baseline; 1.1004x over previous
import jax
import jax.numpy as jnp
from jax import lax
from jax.experimental import pallas as pl
from jax.experimental.pallas import tpu as pltpu

N_DEV = 4
N_HOP = N_DEV - 1
S = 2


def kernel(x, w_mat):
    k, n = w_mat.shape
    m = x.shape[0]
    m_per = m // N_DEV
    n_half = n // 2
    sub = n_half // S

    def body(
        x_ref, w_ref, out_ref,
        send_a, send_b, recv_a, recv_b,
        pstore_a, pstore_b,
        ss_a, rs_a, ss_b, rs_b,
    ):
        my = lax.axis_index("i")
        left = lax.rem(my + (N_DEV - 1), N_DEV)
        right = lax.rem(my + 1, N_DEV)

        barrier_sem = pltpu.get_barrier_semaphore()
        for nbr in (left, right):
            pl.semaphore_signal(
                barrier_sem, inc=1,
                device_id=(nbr,), device_id_type=pl.DeviceIdType.MESH,
            )
        pl.semaphore_wait(barrier_sem, 2)

        w = w_ref[...].astype(jnp.bfloat16)

        def partial(c, lo=None, width=None):
            xs = x_ref[pl.ds(c * m_per, m_per), :].astype(jnp.bfloat16)
            ws = w if lo is None else w[:, lo:lo + width]
            return jnp.dot(xs, ws, preferred_element_type=jnp.float32)

        c_dm1 = lax.rem(my + N_DEV - 1, N_DEV)
        c_dp1 = lax.rem(my + 1, N_DEV)
        c_dp2 = lax.rem(my + 2, N_DEV)

        def mk(h, s, tgt, src, dst, ssem, rsem):
            return pltpu.make_async_remote_copy(
                src_ref=src.at[h, s], dst_ref=dst.at[h, s],
                send_sem=ssem.at[h, s], recv_sem=rsem.at[h, s],
                device_id=(tgt,), device_id_type=pl.DeviceIdType.MESH,
            )

        def mk_a(h, s):
            return mk(h, s, right, send_a, recv_a, ss_a, rs_a)

        def mk_b(h, s):
            return mk(h, s, left, send_b, recv_b, ss_b, rs_b)

        for s in range(S):
            send_a[0, s] = partial(c_dm1, s * sub, sub).astype(jnp.bfloat16)
            mk_a(0, s).start()
            send_b[0, s] = partial(
                c_dp1, n_half + s * sub, sub
            ).astype(jnp.bfloat16)
            mk_b(0, s).start()

        pstore_a[:, :] = partial(c_dp1, 0, n_half).astype(jnp.bfloat16)
        pstore_b[:, :] = partial(c_dm1, n_half, n_half).astype(jnp.bfloat16)
        pc = partial(c_dp2)

        pd = None
        for h in range(N_HOP - 1):
            for s in range(S):
                if h == 0:
                    add_a = pc[:, s * sub:(s + 1) * sub]
                    add_b = pc[:, n_half + s * sub:n_half + (s + 1) * sub]
                else:
                    add_a = pstore_a[:, s * sub:(s + 1) * sub].astype(
                        jnp.float32)
                    add_b = pstore_b[:, s * sub:(s + 1) * sub].astype(
                        jnp.float32)
                mk_a(h, s).wait_recv()
                send_a[h + 1, s] = (
                    recv_a[h, s].astype(jnp.float32) + add_a
                ).astype(jnp.bfloat16)
                mk_a(h + 1, s).start()
                mk_b(h, s).wait_recv()
                send_b[h + 1, s] = (
                    recv_b[h, s].astype(jnp.float32) + add_b
                ).astype(jnp.bfloat16)
                mk_b(h + 1, s).start()
            if h == 0:
                pd = partial(my)

        for s in range(S):
            mk_a(N_HOP - 1, s).wait_recv()
            out_ref[:, s * sub:(s + 1) * sub] = jnp.maximum(
                recv_a[N_HOP - 1, s].astype(jnp.float32)
                + pd[:, s * sub:(s + 1) * sub], 0.0,
            )
            mk_b(N_HOP - 1, s).wait_recv()
            lo = n_half + s * sub
            out_ref[:, lo:lo + sub] = jnp.maximum(
                recv_b[N_HOP - 1, s].astype(jnp.float32)
                + pd[:, lo:lo + sub], 0.0,
            )

        for h in range(N_HOP):
            for s in range(S):
                mk_a(h, s).wait_send()
                mk_b(h, s).wait_send()

    comm = pltpu.VMEM((N_HOP, S, m_per, sub), jnp.bfloat16)
    half = pltpu.VMEM((m_per, n_half), jnp.bfloat16)
    sems = pltpu.SemaphoreType.DMA((N_HOP, S))
    return pl.pallas_call(
        body,
        out_shape=jax.ShapeDtypeStruct((m_per, n), jnp.float32),
        in_specs=[
            pl.BlockSpec(memory_space=pltpu.VMEM),
            pl.BlockSpec(memory_space=pltpu.VMEM),
        ],
        out_specs=pl.BlockSpec(memory_space=pltpu.VMEM),
        scratch_shapes=[comm, comm, comm, comm, half, half,
                        sems, sems, sems, sems],
        compiler_params=pltpu.CompilerParams(collective_id=0),
    )(x, w_mat)
